# baseline (speedup 1.0000x reference)
import functools
import math

import jax
import jax.numpy as jnp
from jax import lax
from jax.experimental import pallas as pl
from jax.experimental.pallas import tpu as pltpu
from jax.experimental.pallas import tpu_sc as plsc

D_MODEL = 1024
SCALE = math.sqrt(D_MODEL)

_INFO = plsc.get_sparse_core_info()
NC, NS, L = _INFO.num_cores, _INFO.num_subcores, _INFO.num_lanes
NW = NC * NS

CH = 32
HH = CH // 2
NG = 2
NSB = 2


def _emb_body(b_per_w, n_chunk, x_hbm, table_hbm, out_hbm,
              idx_v, gbuf, sbuf, gsem, ssem):
    wid = lax.axis_index("s") * NC + lax.axis_index("c")
    base = wid * b_per_w

    pltpu.sync_copy(x_hbm.at[pl.ds(base, b_per_w)], idx_v)

    def gather(c, b):
        return pltpu.make_async_copy(
            table_hbm.at[idx_v.at[pl.ds(c * CH, CH)]],
            gbuf.at[b], gsem.at[b])

    def store_half(j, s):
        return pltpu.make_async_copy(
            sbuf.at[s], out_hbm.at[pl.ds(base + j * HH, HH)], ssem.at[s])

    gather(0, 0).start()
    gather(1, 1).start()

    def group(g, _):
        for c2 in range(2):
            c = 2 * g + c2
            gather(c, c2).wait()
            for h in range(2):
                j = 2 * c + h

                @pl.when(c >= 1)
                def _():
                    store_half(j - NSB, h).wait()

                def scale_row(r, _, c2=c2, h=h):
                    for k in range(D_MODEL // L):
                        sbuf[h, r, pl.ds(k * L, L)] = (
                            gbuf[c2, h * HH + r, pl.ds(k * L, L)] * SCALE)
                    return 0

                lax.fori_loop(0, HH, scale_row, 0, unroll=False)
                store_half(j, h).start()

            @pl.when(c + 2 < n_chunk)
            def _():
                gather(c + 2, c2).start()
        return 0

    lax.fori_loop(0, n_chunk // 2, group, 0, unroll=False)
    store_half(2 * n_chunk - 2, 0).wait()
    store_half(2 * n_chunk - 1, 1).wait()


def kernel(x, table):
    orig_shape = x.shape
    xf = x.reshape(-1).astype(jnp.int32)
    b_total = xf.shape[0]
    b_per_w = b_total // NW
    n_chunk = b_per_w // CH

    mesh = plsc.VectorSubcoreMesh(core_axis_name="c", subcore_axis_name="s")
    k = pl.kernel(
        functools.partial(_emb_body, b_per_w, n_chunk),
        mesh=mesh,
        out_type=jax.ShapeDtypeStruct((b_total, D_MODEL), jnp.float32),
        scratch_types=[
            pltpu.VMEM((b_per_w,), jnp.int32),
            pltpu.VMEM((NG, CH, D_MODEL), jnp.float32),
            pltpu.VMEM((NSB, HH, D_MODEL), jnp.float32),
            pltpu.SemaphoreType.DMA((NG,)),
            pltpu.SemaphoreType.DMA((NSB,)),
        ],
    )
    out = k(xf, table)
    return out.reshape(*orig_shape, D_MODEL)


# speedup vs baseline: 1.1317x; 1.1317x over previous
import functools
import math

import jax
import jax.numpy as jnp
from jax import lax
from jax.experimental import pallas as pl
from jax.experimental.pallas import tpu as pltpu
from jax.experimental.pallas import tpu_sc as plsc

D_MODEL = 1024
SCALE = math.sqrt(D_MODEL)

_INFO = plsc.get_sparse_core_info()
NC, NS, L = _INFO.num_cores, _INFO.num_subcores, _INFO.num_lanes
NW = NC * NS

CH = 32
HH = CH // 2
NBUF = 3


def _emb_body(b_per_w, n_chunk, x_hbm, table_hbm, out_hbm,
              idx_v, rows_v, gsem, ssem):
    wid = lax.axis_index("s") * NC + lax.axis_index("c")
    base = wid * b_per_w

    pltpu.sync_copy(x_hbm.at[pl.ds(base, b_per_w)], idx_v)

    def gather(c):
        return pltpu.make_async_copy(
            table_hbm.at[idx_v.at[pl.ds(c * CH, CH)]],
            rows_v.at[c % NBUF], gsem.at[c % NBUF])

    def store_half(c, h):
        b = c % NBUF
        return pltpu.make_async_copy(
            rows_v.at[b, pl.ds(h * HH, HH)],
            out_hbm.at[pl.ds(base + c * CH + h * HH, HH)], ssem.at[b, h])

    gather(0).start()
    gather(1).start()
    for c in range(n_chunk):
        b = c % NBUF
        gather(c).wait()
        for h in range(2):

            def scale_row(r, _, b=b, h=h):
                for k in range(D_MODEL // L):
                    rows_v[b, h * HH + r, pl.ds(k * L, L)] = (
                        rows_v[b, h * HH + r, pl.ds(k * L, L)] * SCALE)
                return 0

            lax.fori_loop(0, HH, scale_row, 0, unroll=False)
            store_half(c, h).start()
        if c + 2 < n_chunk:
            if c >= 1:
                store_half(c - 1, 0).wait()
                store_half(c - 1, 1).wait()
            gather(c + 2).start()
    for c in range(n_chunk - NBUF, n_chunk):
        store_half(c, 0).wait()
        store_half(c, 1).wait()


def kernel(x, table):
    orig_shape = x.shape
    xf = x.reshape(-1).astype(jnp.int32)
    b_total = xf.shape[0]
    b_per_w = b_total // NW
    n_chunk = b_per_w // CH

    mesh = plsc.VectorSubcoreMesh(core_axis_name="c", subcore_axis_name="s")
    k = pl.kernel(
        functools.partial(_emb_body, b_per_w, n_chunk),
        mesh=mesh,
        out_type=jax.ShapeDtypeStruct((b_total, D_MODEL), jnp.float32),
        scratch_types=[
            pltpu.VMEM((b_per_w,), jnp.int32),
            pltpu.VMEM((NBUF, CH, D_MODEL), jnp.float32),
            pltpu.SemaphoreType.DMA((NBUF,)),
            pltpu.SemaphoreType.DMA((NBUF, 2)),
        ],
    )
    out = k(xf, table)
    return out.reshape(*orig_shape, D_MODEL)
